# baseline (speedup 1.0000x reference)
import jax
import jax.numpy as jnp
from jax import lax
from jax.experimental import pallas as pl

NUM_CLASSES = 1000
S_PAD = 32
C_PAD = 1024
BLOCK_ROWS = 128


def _onehot_body(x_ref, out_ref):
    idx = x_ref[...]
    idx = jnp.concatenate(
        [idx, jnp.full((BLOCK_ROWS, S_PAD - 20), -1, jnp.int32)], axis=1)
    classes = lax.broadcasted_iota(jnp.int32, (BLOCK_ROWS, S_PAD, C_PAD), 2)
    out_ref[...] = (idx[:, :, None] == classes).astype(jnp.int8)


def kernel(x):
    B, S = x.shape
    grid = (B // BLOCK_ROWS,)
    packed = pl.pallas_call(
        _onehot_body,
        grid=grid,
        in_specs=[pl.BlockSpec((BLOCK_ROWS, S), lambda i: (i, 0))],
        out_specs=pl.BlockSpec((BLOCK_ROWS, S_PAD, C_PAD), lambda i: (i, 0, 0)),
        out_shape=jax.ShapeDtypeStruct((B, S_PAD, C_PAD), jnp.int8),
    )(x.astype(jnp.int32))
    return packed[:, :S, :NUM_CLASSES].astype(jnp.float32)


# speedup vs baseline: 1.3228x; 1.3228x over previous
import jax
import jax.numpy as jnp
from jax import lax
from jax.experimental import pallas as pl

NUM_CLASSES = 1000
S_PAD = 24
C_PAD = 1024
BLOCK_ROWS = 128


def _onehot_body(x_ref, out_ref):
    idx = x_ref[...]
    idx = jnp.concatenate(
        [idx, jnp.full((BLOCK_ROWS, S_PAD - 20), -1, jnp.int32)], axis=1)
    classes = lax.broadcasted_iota(jnp.int32, (BLOCK_ROWS, S_PAD, C_PAD), 2)
    out_ref[...] = (idx[:, :, None] == classes).astype(jnp.float32)


def kernel(x):
    B, S = x.shape
    grid = (B // BLOCK_ROWS,)
    padded = pl.pallas_call(
        _onehot_body,
        grid=grid,
        in_specs=[pl.BlockSpec((BLOCK_ROWS, S), lambda i: (i, 0))],
        out_specs=pl.BlockSpec((BLOCK_ROWS, S_PAD, C_PAD), lambda i: (i, 0, 0)),
        out_shape=jax.ShapeDtypeStruct((B, S_PAD, C_PAD), jnp.float32),
    )(x.astype(jnp.int32))
    return padded[:, :S, :NUM_CLASSES]
